# Initial kernel scaffold; baseline (speedup 1.0000x reference)
import functools

import jax
import jax.numpy as jnp
from jax import lax
from jax.experimental import pallas as pl
from jax.experimental.pallas import tpu as pltpu
from jax.experimental.pallas import tpu_sc as plsc

N = 10000
NP = 10240
K = 10
KPAD = 16


_KNN_R = 256


def _knn_body(q_ref, c_ref, o_ref):
    pid = pl.program_id(0)
    q = q_ref[...]
    d2 = jnp.dot(q, c_ref[...], preferred_element_type=jnp.float32)
    cols = lax.broadcasted_iota(jnp.int32, (_KNN_R, NP), 1)
    rows = lax.broadcasted_iota(jnp.int32, (_KNN_R, NP), 0) + pid * _KNN_R
    d2 = jnp.where(cols == rows, jnp.float32(4e30), d2)
    for t in range(K):
        m = jnp.min(d2, axis=1)
        ismin = d2 == m[:, None]
        idx = jnp.min(jnp.where(ismin, cols, jnp.int32(NP)), axis=1)
        o_ref[t, :] = idx
        d2 = jnp.where(cols == idx[:, None], jnp.float32(4e30), d2)


def _knn(qmat, cmat):
    return pl.pallas_call(
        _knn_body,
        grid=(NP // _KNN_R,),
        in_specs=[
            pl.BlockSpec((_KNN_R, 8), lambda i: (i, 0)),
            pl.BlockSpec((8, NP), lambda i: (0, 0)),
        ],
        out_specs=pl.BlockSpec((KPAD, _KNN_R), lambda i: (0, i)),
        out_shape=jax.ShapeDtypeStruct((KPAD, NP), jnp.int32),
    )(qmat, cmat)


_NC, _NS = 2, 16
_NW = _NC * _NS
_QPW = NP // _NW
_QS = 80
_NSUB = _QPW // _QS


def _agg_body(nbr_hbm, z_hbm, out_hbm, gidx, sidx, buf, acc_sh, sem):
    cid = lax.axis_index("c")
    sid = lax.axis_index("s")
    wid = cid * _NS + sid
    base = wid * _QPW
    sbase = sid * _QPW

    pltpu.sync_copy(nbr_hbm.at[pl.ds(0, K), pl.ds(base, _QPW)], gidx)

    for u in range(_NSUB):
        for t in range(_QS // 16):
            sidx[u, pl.ds(t * 16, 16)] = (
                lax.iota(jnp.int32, 16) + (sbase + u * _QS + t * 16))

    for u in range(_NSUB):
        pltpu.async_copy(z_hbm.at[gidx.at[0, pl.ds(u * _QS, _QS)]], buf, sem).wait()
        pltpu.sync_copy(buf, acc_sh.at[pl.ds(sbase + u * _QS, _QS)])

    def j_step(j, carry):
        for u in range(_NSUB):
            pltpu.async_copy(z_hbm.at[gidx.at[j, pl.ds(u * _QS, _QS)]], buf, sem).wait()
            pltpu.sync_copy(buf, acc_sh.at[sidx.at[u]], add=True)
        return carry

    lax.fori_loop(1, K, j_step, 0)

    pltpu.sync_copy(acc_sh.at[pl.ds(sbase, _QPW)], out_hbm.at[pl.ds(base, _QPW)])


def _sc_agg(z, nbr, width):
    mesh = plsc.VectorSubcoreMesh(core_axis_name="c", subcore_axis_name="s")
    fn = pl.kernel(
        _agg_body,
        out_type=jax.ShapeDtypeStruct((NP, width), jnp.float32),
        mesh=mesh,
        scratch_types=[
            pltpu.VMEM((K, _QPW), jnp.int32),
            pltpu.VMEM((_NSUB, _QS), jnp.int32),
            pltpu.VMEM((_QS, width), jnp.float32),
            pltpu.VMEM_SHARED((_NS * _QPW, width), jnp.float32),
            pltpu.SemaphoreType.DMA,
        ],
    )
    return fn(nbr, z)


_ROWS = 1000


def _layer_body(has_agg, x_ref, *refs):
    if has_agg:
        g_ref, w_ref, b_ref, z_ref, s_ref = refs
        x = x_ref[...] + g_ref[...]
    else:
        w_ref, b_ref, z_ref, s_ref = refs
        x = x_ref[...]
    z = jnp.dot(x, w_ref[...], preferred_element_type=jnp.float32)
    z = z + b_ref[0:1, :]
    z = jnp.where(z >= 0, z, jnp.float32(0.33) * z)
    z_ref[...] = z
    cout = z.shape[1]
    s1 = jnp.sum(z, axis=0)[None, :]
    s2 = jnp.sum(z * z, axis=0)[None, :]
    r8 = lax.broadcasted_iota(jnp.int32, (8, cout), 0)
    s8 = jnp.where(r8 == 0, s1, jnp.where(r8 == 1, s2, jnp.float32(0.0)))
    i = pl.program_id(0)

    @pl.when(i == 0)
    def _init():
        s_ref[...] = jnp.zeros((8, cout), jnp.float32)

    s_ref[...] += s8


def _mlp_layer(x, agg, W, b):
    cin, cout = W.shape
    b8 = jnp.broadcast_to(b[None, :], (8, cout))
    ins = [x] + ([agg] if agg is not None else []) + [W, b8]
    in_specs = [pl.BlockSpec((_ROWS, cin), lambda i: (i, 0))]
    if agg is not None:
        in_specs.append(pl.BlockSpec((_ROWS, cin), lambda i: (i, 0)))
    in_specs += [
        pl.BlockSpec((cin, cout), lambda i: (0, 0)),
        pl.BlockSpec((8, cout), lambda i: (0, 0)),
    ]
    return pl.pallas_call(
        functools.partial(_layer_body, agg is not None),
        grid=(N // _ROWS,),
        in_specs=in_specs,
        out_specs=[
            pl.BlockSpec((_ROWS, cout), lambda i: (i, 0)),
            pl.BlockSpec((8, cout), lambda i: (0, 0)),
        ],
        out_shape=[
            jax.ShapeDtypeStruct((N, cout), jnp.float32),
            jax.ShapeDtypeStruct((8, cout), jnp.float32),
        ],
    )(*ins)


def _final_body(x_ref, w_ref, b_ref, o_ref):
    z = jnp.dot(x_ref[...], w_ref[...], preferred_element_type=jnp.float32)
    o_ref[...] = z + b_ref[0:1, :]


def _final_layer(x, W, b):
    cin, cout = W.shape
    b8 = jnp.broadcast_to(b[None, :], (8, cout))
    return pl.pallas_call(
        _final_body,
        grid=(N // _ROWS,),
        in_specs=[
            pl.BlockSpec((_ROWS, cin), lambda i: (i, 0)),
            pl.BlockSpec((cin, cout), lambda i: (0, 0)),
            pl.BlockSpec((8, cout), lambda i: (0, 0)),
        ],
        out_specs=pl.BlockSpec((_ROWS, cout), lambda i: (i, 0)),
        out_shape=jax.ShapeDtypeStruct((N, cout), jnp.float32),
    )(x, W, b8)


def _stats_to_affine(sums, g, be):
    m = sums[0] / N
    v = sums[1] / N - m * m
    a = g / jnp.sqrt(v + 1e-5)
    return a, be - m * a


def kernel(input, params):
    pc = input
    coords = pc[:, 0:3]
    sq = jnp.sum(coords * coords, axis=1)

    zcol = jnp.zeros((N,), jnp.float32)
    qmat = jnp.stack(
        [-2.0 * coords[:, 0], -2.0 * coords[:, 1], -2.0 * coords[:, 2],
         jnp.ones((N,), jnp.float32), sq, zcol, zcol, zcol], axis=1)
    qmat = jnp.pad(qmat, ((0, NP - N), (0, 0)))
    cmat = jnp.stack(
        [jnp.pad(coords[:, 0], (0, NP - N)),
         jnp.pad(coords[:, 1], (0, NP - N)),
         jnp.pad(coords[:, 2], (0, NP - N)),
         jnp.pad(sq, (0, NP - N), constant_values=1e30),
         jnp.pad(jnp.ones((N,), jnp.float32), (0, NP - N)),
         jnp.zeros((NP,), jnp.float32),
         jnp.zeros((NP,), jnp.float32),
         jnp.zeros((NP,), jnp.float32)], axis=0)

    nbr = _knn(qmat, cmat)

    nc = (coords - 384.0) / 384.0
    x0 = jnp.concatenate([nc, pc[:, 4:5], jnp.zeros((N, 12), jnp.float32)], axis=1)

    z = x0
    a = jnp.ones((16,), jnp.float32)
    c = jnp.zeros((16,), jnp.float32)

    for name in ("gin1", "gin2", "gin3"):
        layers = params[name]
        aggz = _sc_agg(z, nbr, z.shape[1])[:N]
        for li, (W, b, g, be) in enumerate(layers):
            if name == "gin1" and li == 0:
                W = jnp.pad(W, ((0, 12), (0, 0)))
            Wf = a[:, None] * W
            if li == 0:
                bf = b + (1.0 + K) * (c @ W)
                z, sums = _mlp_layer(z, aggz, Wf, bf)
            else:
                bf = b + c @ W
                z, sums = _mlp_layer(z, None, Wf, bf)
            a, c = _stats_to_affine(sums, g, be)

    for (W, b, g, be) in params["mlp3"]:
        Wf = a[:, None] * W
        bf = b + c @ W
        z, sums = _mlp_layer(z, None, Wf, bf)
        a, c = _stats_to_affine(sums, g, be)

    Wfin, bfin = params["final"]
    return _final_layer(z, a[:, None] * Wfin, bfin + c @ Wfin)


# speedup vs baseline: 5.7311x; 5.7311x over previous
import functools

import jax
import jax.numpy as jnp
from jax import lax
from jax.experimental import pallas as pl
from jax.experimental.pallas import tpu as pltpu
from jax.experimental.pallas import tpu_sc as plsc

N = 10000
NP = 10240
K = 10
KPAD = 16


_KNN_R = 256


def _knn_body(q_ref, c_ref, sqq_ref, sqc_ref, o_ref):
    pid = pl.program_id(0)
    dot = jnp.dot(q_ref[...], c_ref[...], preferred_element_type=jnp.float32)
    d2 = (sqq_ref[...] + sqc_ref[0:1, :]) - 2.0 * dot
    cols = lax.broadcasted_iota(jnp.int32, (_KNN_R, NP), 1)
    rows = lax.broadcasted_iota(jnp.int32, (_KNN_R, NP), 0) + pid * _KNN_R
    d2 = jnp.where(cols == rows, jnp.float32(4e30), d2)
    for t in range(K):
        m = jnp.min(d2, axis=1)
        ismin = d2 == m[:, None]
        idx = jnp.min(jnp.where(ismin, cols, jnp.int32(NP)), axis=1)
        o_ref[t, :] = idx
        d2 = jnp.where(cols == idx[:, None], jnp.float32(4e30), d2)


def _knn(qb, cb, sqq, sqc8):
    return pl.pallas_call(
        _knn_body,
        grid=(NP // _KNN_R,),
        in_specs=[
            pl.BlockSpec((_KNN_R, 8), lambda i: (i, 0)),
            pl.BlockSpec((8, NP), lambda i: (0, 0)),
            pl.BlockSpec((_KNN_R, 1), lambda i: (i, 0)),
            pl.BlockSpec((8, NP), lambda i: (0, 0)),
        ],
        out_specs=pl.BlockSpec((KPAD, _KNN_R), lambda i: (0, i)),
        out_shape=jax.ShapeDtypeStruct((KPAD, NP), jnp.int32),
    )(qb, cb, sqq, sqc8)


_NC, _NS = 2, 16
_NW = _NC * _NS
_QPW = NP // _NW
_QS = 80
_NSUB = _QPW // _QS


def _agg_body(nbr_hbm, z_hbm, out_hbm, gidx, sidx, buf, acc_sh, sem):
    cid = lax.axis_index("c")
    sid = lax.axis_index("s")
    wid = cid * _NS + sid
    base = wid * _QPW
    sbase = sid * _QPW

    for j in range(K):
        pltpu.sync_copy(nbr_hbm.at[pl.ds(j * NP + base, _QPW)], gidx.at[j])

    for u in range(_NSUB):
        for t in range(_QS // 16):
            sidx[u, pl.ds(t * 16, 16)] = (
                lax.iota(jnp.int32, 16) + (sbase + u * _QS + t * 16))

    for u in range(_NSUB):
        pltpu.async_copy(z_hbm.at[gidx.at[0, pl.ds(u * _QS, _QS)]], buf, sem).wait()
        pltpu.sync_copy(buf, acc_sh.at[pl.ds(sbase + u * _QS, _QS)])

    def j_step(j, carry):
        for u in range(_NSUB):
            pltpu.async_copy(z_hbm.at[gidx.at[j, pl.ds(u * _QS, _QS)]], buf, sem).wait()
            pltpu.sync_copy(buf, acc_sh.at[sidx.at[u]], add=True)
        return carry

    lax.fori_loop(1, K, j_step, 0)

    pltpu.sync_copy(acc_sh.at[pl.ds(sbase, _QPW)], out_hbm.at[pl.ds(base, _QPW)])


def _sc_agg(z, nbr, width):
    mesh = plsc.VectorSubcoreMesh(core_axis_name="c", subcore_axis_name="s")
    fn = pl.kernel(
        _agg_body,
        out_type=jax.ShapeDtypeStruct((NP, width), jnp.float32),
        mesh=mesh,
        scratch_types=[
            pltpu.VMEM((KPAD, _QPW), jnp.int32),
            pltpu.VMEM((_NSUB, _QS), jnp.int32),
            pltpu.VMEM((_QS, width), jnp.float32),
            pltpu.VMEM_SHARED((_NS * _QPW, width), jnp.float32),
            pltpu.SemaphoreType.DMA,
        ],
        compiler_params=pltpu.CompilerParams(use_tc_tiling_on_sc=False),
    )
    return fn(nbr.reshape(-1), z)


_ROWS = 1000


def _layer_body(has_agg, x_ref, *refs):
    if has_agg:
        g_ref, a_ref, c_ref, w_ref, b_ref, z_ref, s_ref = refs
        x = x_ref[...] + g_ref[...]
    else:
        a_ref, c_ref, w_ref, b_ref, z_ref, s_ref = refs
        x = x_ref[...]
    x = a_ref[0:1, :] * x + c_ref[0:1, :]
    z = jnp.dot(x.astype(jnp.bfloat16), w_ref[...],
                preferred_element_type=jnp.float32)
    z = z + b_ref[0:1, :]
    z = jnp.where(z >= 0, z, jnp.float32(0.33) * z)
    z_ref[...] = z
    cout = z.shape[1]
    s1 = jnp.sum(z, axis=0)[None, :]
    s2 = jnp.sum(z * z, axis=0)[None, :]
    r8 = lax.broadcasted_iota(jnp.int32, (8, cout), 0)
    s8 = jnp.where(r8 == 0, s1, jnp.where(r8 == 1, s2, jnp.float32(0.0)))
    i = pl.program_id(0)

    @pl.when(i == 0)
    def _init():
        s_ref[...] = jnp.zeros((8, cout), jnp.float32)

    s_ref[...] += s8


def _mlp_layer(x, agg, a, c, W, b):
    cin, cout = W.shape
    a8 = jnp.broadcast_to(a[None, :], (8, cin))
    c8 = jnp.broadcast_to(c[None, :], (8, cin))
    b8 = jnp.broadcast_to(b[None, :], (8, cout))
    wb = W.astype(jnp.bfloat16)
    ins = [x] + ([agg] if agg is not None else []) + [a8, c8, wb, b8]
    in_specs = [pl.BlockSpec((_ROWS, cin), lambda i: (i, 0))]
    if agg is not None:
        in_specs.append(pl.BlockSpec((_ROWS, cin), lambda i: (i, 0)))
    in_specs += [
        pl.BlockSpec((8, cin), lambda i: (0, 0)),
        pl.BlockSpec((8, cin), lambda i: (0, 0)),
        pl.BlockSpec((cin, cout), lambda i: (0, 0)),
        pl.BlockSpec((8, cout), lambda i: (0, 0)),
    ]
    return pl.pallas_call(
        functools.partial(_layer_body, agg is not None),
        grid=(N // _ROWS,),
        in_specs=in_specs,
        out_specs=[
            pl.BlockSpec((_ROWS, cout), lambda i: (i, 0)),
            pl.BlockSpec((8, cout), lambda i: (0, 0)),
        ],
        out_shape=[
            jax.ShapeDtypeStruct((N, cout), jnp.float32),
            jax.ShapeDtypeStruct((8, cout), jnp.float32),
        ],
    )(*ins)


def _final_body(x_ref, a_ref, c_ref, w_ref, b_ref, o_ref):
    x = a_ref[0:1, :] * x_ref[...] + c_ref[0:1, :]
    z = jnp.dot(x.astype(jnp.bfloat16), w_ref[...],
                preferred_element_type=jnp.float32)
    o_ref[...] = z + b_ref[0:1, :]


def _final_layer(x, a, c, W, b):
    cin, cout = W.shape
    a8 = jnp.broadcast_to(a[None, :], (8, cin))
    c8 = jnp.broadcast_to(c[None, :], (8, cin))
    b8 = jnp.broadcast_to(b[None, :], (8, cout))
    return pl.pallas_call(
        _final_body,
        grid=(N // _ROWS,),
        in_specs=[
            pl.BlockSpec((_ROWS, cin), lambda i: (i, 0)),
            pl.BlockSpec((8, cin), lambda i: (0, 0)),
            pl.BlockSpec((8, cin), lambda i: (0, 0)),
            pl.BlockSpec((cin, cout), lambda i: (0, 0)),
            pl.BlockSpec((8, cout), lambda i: (0, 0)),
        ],
        out_specs=pl.BlockSpec((_ROWS, cout), lambda i: (i, 0)),
        out_shape=jax.ShapeDtypeStruct((N, cout), jnp.float32),
    )(x, a8, c8, W.astype(jnp.bfloat16), b8)


def _stats_to_affine(sums, g, be):
    m = sums[0] / N
    v = sums[1] / N - m * m
    a = g / jnp.sqrt(v + 1e-5)
    return a, be - m * a


def kernel(input, params):
    pc = input
    coords = pc[:, 0:3]
    sq = jnp.sum(coords * coords, axis=1)

    cpad = jnp.pad(coords, ((0, NP - N), (0, 0)))
    qb = jnp.pad(cpad, ((0, 0), (0, 5))).astype(jnp.bfloat16)
    cb = jnp.pad(cpad.T, ((0, 5), (0, 0))).astype(jnp.bfloat16)
    sqq = jnp.pad(sq, (0, NP - N))[:, None]
    sqc8 = jnp.broadcast_to(
        jnp.pad(sq, (0, NP - N), constant_values=1e30)[None, :], (8, NP))

    nbr = _knn(qb, cb, sqq, sqc8)

    nc = (coords - 384.0) / 384.0
    x0 = jnp.concatenate([nc, pc[:, 4:5], jnp.zeros((N, 12), jnp.float32)], axis=1)

    z = x0
    a = jnp.ones((16,), jnp.float32)
    c = jnp.zeros((16,), jnp.float32)

    for name in ("gin1", "gin2", "gin3"):
        layers = params[name]
        aggz = _sc_agg(z, nbr, z.shape[1])[:N]
        for li, (W, b, g, be) in enumerate(layers):
            if name == "gin1" and li == 0:
                W = jnp.pad(W, ((0, 12), (0, 0)))
            if li == 0:
                z, sums = _mlp_layer(z, aggz, a, (1.0 + K) * c, W, b)
            else:
                z, sums = _mlp_layer(z, None, a, c, W, b)
            a, c = _stats_to_affine(sums, g, be)

    for (W, b, g, be) in params["mlp3"]:
        z, sums = _mlp_layer(z, None, a, c, W, b)
        a, c = _stats_to_affine(sums, g, be)

    Wfin, bfin = params["final"]
    return _final_layer(z, a, c, Wfin, bfin)
